# baseline (speedup 1.0000x reference)
import functools

import jax
import jax.numpy as jnp
from jax.experimental import pallas as pl
from jax.experimental.pallas import tpu as pltpu

_BM = 128


@functools.cache
def _gumbel_const(batch: int, n: int):
    return jax.random.gumbel(jax.random.key(42), (batch, n), dtype=jnp.float32)


def _body(x_ref, g_ref, w1_ref, b1_ref, w2_ref, b2_ref, o_ref):
    x = x_ref[...]
    y = x * x
    n = y.shape[1] // 2
    t = y[:, :n] + y[:, n:] + g_ref[...]
    mx = jnp.max(t, axis=1, keepdims=True)
    e = jnp.exp(2.0 * (t - mx))
    s = jnp.sum(e, axis=1, keepdims=True)
    num = jnp.dot(e, w1_ref[...], preferred_element_type=jnp.float32)
    h = jnp.maximum(num / s + b1_ref[...], 0.0)
    o_ref[...] = (
        jnp.dot(h, w2_ref[...], preferred_element_type=jnp.float32) + b2_ref[...]
    )


def kernel(quantum_state, W1, b1, W2, b2):
    batch, n, _ = quantum_state.shape
    odim = W2.shape[1]
    xp = jnp.transpose(quantum_state, (0, 2, 1)).reshape(batch, 2 * n)
    g = _gumbel_const(batch, n)
    bm = min(_BM, batch)

    return pl.pallas_call(
        _body,
        grid=(batch // bm,),
        in_specs=[
            pl.BlockSpec((bm, 2 * n), lambda i: (i, 0)),
            pl.BlockSpec((bm, n), lambda i: (i, 0)),
            pl.BlockSpec((n, W1.shape[1]), lambda i: (0, 0)),
            pl.BlockSpec((1, W1.shape[1]), lambda i: (0, 0)),
            pl.BlockSpec(W2.shape, lambda i: (0, 0)),
            pl.BlockSpec((1, odim), lambda i: (0, 0)),
        ],
        out_specs=pl.BlockSpec((bm, odim), lambda i: (i, 0)),
        out_shape=jax.ShapeDtypeStruct((batch, odim), jnp.float32),
        compiler_params=pltpu.CompilerParams(
            dimension_semantics=("parallel",),
        ),
    )(xp, g, W1, b1.reshape(1, -1), W2, b2.reshape(1, -1))


# speedup vs baseline: 1.1395x; 1.1395x over previous
import functools

import jax
import jax.numpy as jnp
from jax.experimental import pallas as pl
from jax.experimental.pallas import tpu as pltpu

_BM = 64
_LC = 128


@functools.cache
def _gumbel_const(batch: int, n: int):
    with jax.ensure_compile_time_eval():
        g = jax.random.gumbel(jax.random.key(42), (batch, n), dtype=jnp.float32)
        return g.reshape(batch, n // _LC, _LC)


def _body(x_ref, g_ref, w1q_ref, b1_ref, w2_ref, b2_ref, o_ref):
    bm, rows, lc = x_ref.shape
    nc = rows // 2
    x4 = x_ref[...].reshape(bm, nc, 2, lc)
    re = x4[:, :, 0, :]
    im = x4[:, :, 1, :]
    t = re * re + im * im + g_ref[...]
    mx = jnp.max(t, axis=(1, 2), keepdims=True)
    e = jnp.exp(2.0 * (t - mx))
    w1q = w1q_ref[...]
    na = jnp.dot(e[:, 0, :], w1q[0], preferred_element_type=jnp.float32)
    for c in range(1, nc):
        na = na + jnp.dot(e[:, c, :], w1q[c], preferred_element_type=jnp.float32)
    num = na[:, :-1]
    s = na[:, -1:]
    h = jnp.maximum(num / s + b1_ref[...], 0.0)
    o_ref[...] = (
        jnp.dot(h, w2_ref[...], preferred_element_type=jnp.float32) + b2_ref[...]
    )


def kernel(quantum_state, W1, b1, W2, b2):
    batch, n, _ = quantum_state.shape
    odim = W2.shape[1]
    hdim = W1.shape[1]
    nc = n // _LC
    g = _gumbel_const(batch, n)
    w1a = jnp.concatenate([W1, jnp.ones((n, 1), W1.dtype)], axis=1)
    w1q = w1a.reshape(nc, _LC, hdim + 1)
    x3 = (
        quantum_state.reshape(batch, nc, _LC, 2)
        .transpose(0, 1, 3, 2)
        .reshape(batch, 2 * nc, _LC)
    )
    bm = min(_BM, batch)

    return pl.pallas_call(
        _body,
        grid=(batch // bm,),
        in_specs=[
            pl.BlockSpec((bm, 2 * nc, _LC), lambda i: (i, 0, 0)),
            pl.BlockSpec((bm, nc, _LC), lambda i: (i, 0, 0)),
            pl.BlockSpec((nc, _LC, hdim + 1), lambda i: (0, 0, 0)),
            pl.BlockSpec((1, hdim), lambda i: (0, 0)),
            pl.BlockSpec(W2.shape, lambda i: (0, 0)),
            pl.BlockSpec((1, odim), lambda i: (0, 0)),
        ],
        out_specs=pl.BlockSpec((bm, odim), lambda i: (i, 0)),
        out_shape=jax.ShapeDtypeStruct((batch, odim), jnp.float32),
        compiler_params=pltpu.CompilerParams(
            dimension_semantics=("parallel",),
        ),
    )(x3, g, w1q, b1.reshape(1, -1), W2, b2.reshape(1, -1))


# speedup vs baseline: 2.0545x; 1.8030x over previous
import functools

import jax
import jax.numpy as jnp
from jax.experimental import pallas as pl
from jax.experimental.pallas import tpu as pltpu

_BM = 128


@functools.cache
def _gumbel_const(batch: int, n: int):
    with jax.ensure_compile_time_eval():
        return jax.random.gumbel(
            jax.random.key(42), (batch, n), dtype=jnp.float32
        )


def _body(x_ref, g_ref, w1a_ref, b1_ref, w2_ref, b2_ref, o_ref):
    x = x_ref[...]
    y = x * x
    n = y.shape[1] // 2
    t = y[:, :n] + y[:, n:] + g_ref[...]
    mx = jnp.max(t, axis=1, keepdims=True)
    e = jnp.exp(2.0 * (t - mx))
    na = jnp.dot(e, w1a_ref[...], preferred_element_type=jnp.float32)
    num = na[:, :-1]
    s = na[:, -1:]
    h = jnp.maximum(num / s + b1_ref[...], 0.0)
    o_ref[...] = (
        jnp.dot(h, w2_ref[...], preferred_element_type=jnp.float32) + b2_ref[...]
    )


def kernel(quantum_state, W1, b1, W2, b2):
    batch, n, _ = quantum_state.shape
    odim = W2.shape[1]
    hdim = W1.shape[1]
    xp = jnp.transpose(quantum_state, (0, 2, 1)).reshape(batch, 2 * n)
    g = _gumbel_const(batch, n)
    w1a = jnp.concatenate([W1, jnp.ones((n, 1), W1.dtype)], axis=1)
    bm = min(_BM, batch)

    return pl.pallas_call(
        _body,
        grid=(batch // bm,),
        in_specs=[
            pl.BlockSpec((bm, 2 * n), lambda i: (i, 0)),
            pl.BlockSpec((bm, n), lambda i: (i, 0)),
            pl.BlockSpec((n, hdim + 1), lambda i: (0, 0)),
            pl.BlockSpec((1, hdim), lambda i: (0, 0)),
            pl.BlockSpec(W2.shape, lambda i: (0, 0)),
            pl.BlockSpec((1, odim), lambda i: (0, 0)),
        ],
        out_specs=pl.BlockSpec((bm, odim), lambda i: (i, 0)),
        out_shape=jax.ShapeDtypeStruct((batch, odim), jnp.float32),
        compiler_params=pltpu.CompilerParams(
            dimension_semantics=("parallel",),
        ),
    )(xp, g, w1a, b1.reshape(1, -1), W2, b2.reshape(1, -1))
